# baseline (speedup 1.0000x reference)
import functools

import jax
import jax.numpy as jnp
from jax import lax
from jax.experimental import pallas as pl
from jax.experimental.pallas import tpu as pltpu
from jax.experimental.pallas import tpu_sc as plsc

PAD = 0

NC = 2
NS = 16
NW = NC * NS

CH = 128
NB = 4


@functools.lru_cache(maxsize=None)
def _make_gather(n_tok: int, vocab: int, d: int):
    per_w = n_tok // NW
    n_chunk = per_w // CH
    n_group = n_chunk // NB
    mesh = plsc.VectorSubcoreMesh(core_axis_name="c", subcore_axis_name="s")

    @functools.partial(
        pl.kernel,
        mesh=mesh,
        compiler_params=pltpu.CompilerParams(use_tc_tiling_on_sc=False),
        out_type=jax.ShapeDtypeStruct((n_tok, d), jnp.float32),
        scratch_types=[
            pltpu.VMEM((n_chunk, CH), jnp.int32),
            pltpu.VMEM((NB, CH, d), jnp.float32),
            pltpu.SemaphoreType.DMA,
            pltpu.SemaphoreType.DMA,
        ],
    )
    def gather_kernel(tok_hbm, table_hbm, out_hbm, idx_v, rows_v, gsem, osem):
        wid = lax.axis_index("s") * NC + lax.axis_index("c")
        base = wid * per_w
        pltpu.sync_copy(tok_hbm.at[wid], idx_v)

        def group(g, carry):
            j0 = g * NB
            gathers = []
            for b in range(NB):
                gathers.append(
                    pltpu.async_copy(
                        table_hbm.at[idx_v.at[j0 + b]], rows_v.at[b], gsem
                    )
                )
            for c in gathers:
                c.wait()
            outs = []
            for b in range(NB):
                outs.append(
                    pltpu.async_copy(
                        rows_v.at[b],
                        out_hbm.at[pl.ds(base + (j0 + b) * CH, CH)],
                        osem,
                    )
                )
            for c in outs:
                c.wait()
            return carry

        lax.fori_loop(0, n_group, group, 0)

    return gather_kernel


def kernel(tokens, table):
    b, l = tokens.shape
    vocab, d = table.shape
    n_tok = b * l
    tok_grouped = tokens.reshape(NW, (n_tok // NW) // CH, CH)
    feat = _make_gather(n_tok, vocab, d)(tok_grouped, table)
    features = feat.reshape(b, l, d)
    padding_masks = (tokens == PAD)[:, None, None, :]
    sequential_masks = jnp.triu(jnp.ones((l, l), dtype=bool), k=1)
    return features, padding_masks, sequential_masks


# speedup vs baseline: 1.0273x; 1.0273x over previous
import functools

import jax
import jax.numpy as jnp
from jax import lax
from jax.experimental import pallas as pl
from jax.experimental.pallas import tpu as pltpu
from jax.experimental.pallas import tpu_sc as plsc

PAD = 0

NC = 2
NS = 16
NW = NC * NS

CH = 128
NB = 4


@functools.lru_cache(maxsize=None)
def _make_gather(n_tok: int, vocab: int, d: int):
    per_w = n_tok // NW
    n_chunk = per_w // CH
    n_group = n_chunk // NB
    mesh = plsc.VectorSubcoreMesh(core_axis_name="c", subcore_axis_name="s")

    @functools.partial(
        pl.kernel,
        mesh=mesh,
        compiler_params=pltpu.CompilerParams(use_tc_tiling_on_sc=False),
        out_type=jax.ShapeDtypeStruct((n_tok, d), jnp.float32),
        scratch_types=[
            pltpu.VMEM((n_chunk, CH), jnp.int32),
            pltpu.VMEM((2, NB * CH, d), jnp.float32),
            pltpu.SemaphoreType.DMA,
            pltpu.SemaphoreType.DMA,
            pltpu.SemaphoreType.DMA,
        ],
    )
    def gather_kernel(tok_hbm, table_hbm, out_hbm, idx_v, rows_v, g0sem, g1sem, osem):
        wid = lax.axis_index("s") * NC + lax.axis_index("c")
        base = wid * per_w
        gch = NB * CH
        gsems = (g0sem, g1sem)
        pltpu.sync_copy(tok_hbm.at[wid], idx_v)

        def fire_gathers(g, buf):
            for b in range(NB):
                pltpu.async_copy(
                    table_hbm.at[idx_v.at[g * NB + b]],
                    rows_v.at[buf, pl.ds(b * CH, CH)],
                    gsems[buf],
                )

        def wait_gathers(buf):
            pltpu.make_async_copy(
                out_hbm.at[pl.ds(0, gch)], rows_v.at[buf], gsems[buf]
            ).wait()

        def fire_out(g, buf):
            pltpu.async_copy(
                rows_v.at[buf], out_hbm.at[pl.ds(base + g * gch, gch)], osem
            )

        def wait_out():
            pltpu.make_async_copy(
                out_hbm.at[pl.ds(0, gch)], rows_v.at[0], osem
            ).wait()

        fire_gathers(0, 0)
        fire_gathers(1, 1)

        def step(t, carry, last):
            for buf in (0, 1):
                g = 2 * t + buf
                wait_gathers(buf)
                fire_out(g, buf)
                wait_out()
                if not last:
                    fire_gathers(g + 2, buf)
            return carry

        lax.fori_loop(0, n_group // 2 - 1, lambda t, c: step(t, c, False), 0)
        step(n_group // 2 - 1, 0, True)

    return gather_kernel


def kernel(tokens, table):
    b, l = tokens.shape
    vocab, d = table.shape
    n_tok = b * l
    tok_grouped = tokens.reshape(NW, (n_tok // NW) // CH, CH)
    feat = _make_gather(n_tok, vocab, d)(tok_grouped, table)
    features = feat.reshape(b, l, d)
    padding_masks = (tokens == PAD)[:, None, None, :]
    sequential_masks = jnp.triu(jnp.ones((l, l), dtype=bool), k=1)
    return features, padding_masks, sequential_masks
